# baseline (speedup 1.0000x reference)
import functools

import jax
import jax.numpy as jnp
from jax import lax
from jax.experimental import pallas as pl
from jax.experimental.pallas import tpu as pltpu
from jax.experimental.pallas import tpu_sc as plsc

N_NODES = 10000
D = 128
N_EDGES = 320000

NC = 2
NS = 16
NW = NC * NS
EPW = N_EDGES // NW
K = 64
CH = -(-EPW // K)
EPW_PAD = CH * K
N_ACC = 10240
PAD_DST = N_ACC - 1
NZT = 10
NROW_T = N_NODES // NZT
N_PAD = 10240
CPT = N_PAD // NS


def _sc_agg_body(with_cnt, *refs):
    if with_cnt:
        (h_hbm, pck3, outp, outc,
         acc_sh, cnt_sh, pck_all, src_a, dst_a, src_b, dst_b, src_c, dst_c,
         rows_a, rows_b, rows_c, ones_v, cbuf,
         sem_a, sem_b, sem_c, sem_s) = refs
    else:
        (h_hbm, pck3, outp,
         acc_sh, pck_all, src_a, dst_a, src_b, dst_b, src_c, dst_c,
         rows_a, rows_b, rows_c,
         sem_a, sem_b, sem_c, sem_s) = refs
    c = lax.axis_index("c")
    s = lax.axis_index("s")
    w = c * NS + s

    pltpu.sync_copy(pck3.at[w], pck_all)

    z16 = jnp.zeros((16,), jnp.float32)
    m16 = jnp.full((16,), 0xFFFF, jnp.int32)
    s16 = jnp.full((16,), 16, jnp.int32)

    def unpack(kk, src_v, dst_v):
        for j in range(K // 16):
            v = pck_all[kk, pl.ds(j * 16, 16)]
            src_v[pl.ds(j * 16, 16)] = v & m16
            dst_v[pl.ds(j * 16, 16)] = lax.shift_right_logical(v, s16)

    def _zrow(i, _):
        for j in range(8):
            rows_a[i, pl.ds(j * 16, 16)] = z16
        return 0
    lax.fori_loop(0, K, _zrow, 0)
    r0 = s * NROW_T

    @pl.when(s < NZT)
    def _zero_acc():
        for t in range(NROW_T // K):
            pltpu.sync_copy(rows_a, acc_sh.at[pl.ds(r0 + t * K, K)])
        pltpu.sync_copy(rows_a, acc_sh.at[pl.ds(r0 + NROW_T - K, K)])

    @pl.when(s == NZT)
    def _zero_garbage():
        for t in range((N_ACC - N_NODES) // K):
            pltpu.sync_copy(
                rows_a, acc_sh.at[pl.ds(N_NODES + t * K, K)])
        pltpu.sync_copy(rows_a, acc_sh.at[pl.ds(N_ACC - K, K)])

    if with_cnt:
        o16 = jnp.ones((16,), jnp.float32)
        for i in range(K // 16):
            ones_v[pl.ds(i * 16, 16)] = o16

        def _zc(i, _):
            cbuf[pl.ds(i * 16, 16)] = z16
            return 0
        lax.fori_loop(0, CPT // 16, _zc, 0)
        pltpu.sync_copy(cbuf, cnt_sh.at[pl.ds(s * CPT, CPT)])

    plsc.subcore_barrier()

    def g_start(src_v, buf, sem):
        pltpu.async_copy(h_hbm.at[src_v], buf, sem)

    def g_wait(src_v, buf, sem):
        pltpu.make_async_copy(h_hbm.at[src_v], buf, sem).wait()

    def s_start(dst_v, buf):
        pltpu.async_copy(buf, acc_sh.at[dst_v], sem_s, add=True)
        if with_cnt:
            pltpu.sync_copy(ones_v, cnt_sh.at[dst_v], add=True)

    def s_wait(dst_v, buf):
        pltpu.make_async_copy(buf, acc_sh.at[dst_v], sem_s).wait()

    B = ((src_a, dst_a, rows_a, sem_a),
         (src_b, dst_b, rows_b, sem_b),
         (src_c, dst_c, rows_c, sem_c))

    def step(m, do_swait=True, start_next=True):
        sv, dv, buf, sem = B[m % 3]
        g_wait(sv, buf, sem)
        if do_swait:
            pv, pd, pbuf, _ = B[(m - 1) % 3]
            s_wait(pd, pbuf)
        s_start(dv, buf)
        if start_next:
            nv, nd, nbuf, nsem = B[(m + 2) % 3]
            unpack(m + 2, nv, nd)
            g_start(nv, nbuf, nsem)

    unpack(0, src_a, dst_a)
    g_start(src_a, rows_a, sem_a)
    unpack(1, src_b, dst_b)
    g_start(src_b, rows_b, sem_b)
    step(0, do_swait=False)
    step(1)

    def _body(t, _):
        m0 = 2 + 3 * t
        for j in range(3):
            m = m0 + j
            sv, dv, buf, sem = B[(2 + j) % 3]
            g_wait(sv, buf, sem)
            pv, pd, pbuf, _ = B[(1 + j) % 3]
            s_wait(pd, pbuf)
            s_start(dv, buf)
            nv, nd, nbuf, nsem = B[(4 + j) % 3]
            unpack(m + 2, nv, nd)
            g_start(nv, nbuf, nsem)
        return 0
    lax.fori_loop(0, (CH - 4) // 3, _body, 0)

    step(CH - 2, start_next=False)
    step(CH - 1, start_next=False)
    sv, dv, buf, _ = B[(CH - 1) % 3]
    s_wait(dv, buf)

    plsc.subcore_barrier()

    @pl.when(s < NZT)
    def _pub_acc():
        pltpu.sync_copy(acc_sh.at[pl.ds(r0, NROW_T)],
                        outp.at[pl.ds(c * N_NODES + r0, NROW_T)])
    if with_cnt:
        pltpu.sync_copy(cnt_sh.at[pl.ds(s * CPT, CPT)],
                        outc.at[pl.ds(c * N_PAD + s * CPT, CPT)])


def _make_sc_agg(with_cnt):
    out_type = [jax.ShapeDtypeStruct((NC * N_NODES, D), jnp.float32)]
    scratch = [
        pltpu.VMEM_SHARED((N_ACC, D), jnp.float32),
    ]
    if with_cnt:
        out_type.append(jax.ShapeDtypeStruct((NC * N_PAD,), jnp.float32))
        scratch.append(pltpu.VMEM_SHARED((N_PAD,), jnp.float32))
    scratch += [
        pltpu.VMEM((CH, K), jnp.int32),
        pltpu.VMEM((K,), jnp.int32),
        pltpu.VMEM((K,), jnp.int32),
        pltpu.VMEM((K,), jnp.int32),
        pltpu.VMEM((K,), jnp.int32),
        pltpu.VMEM((K,), jnp.int32),
        pltpu.VMEM((K,), jnp.int32),
        pltpu.VMEM((K, D), jnp.float32),
        pltpu.VMEM((K, D), jnp.float32),
        pltpu.VMEM((K, D), jnp.float32),
    ]
    if with_cnt:
        scratch.append(pltpu.VMEM((K,), jnp.float32))
        scratch.append(pltpu.VMEM((CPT,), jnp.float32))
    scratch += [pltpu.SemaphoreType.DMA] * 4
    mesh = plsc.VectorSubcoreMesh(core_axis_name="c", subcore_axis_name="s")
    return pl.kernel(
        functools.partial(_sc_agg_body, with_cnt),
        out_type=out_type if with_cnt else out_type[0],
        mesh=mesh,
        scratch_types=scratch,
        name="sage_sc_agg_cnt" if with_cnt else "sage_sc_agg",
    )


_sc_agg_with_cnt = _make_sc_agg(True)
_sc_agg_no_cnt = _make_sc_agg(False)

ROWS_B = 5000


def _tc_dense_body(h_ref, aggp_ref, cntp_ref, wl_ref, bl_ref, wr_ref,
                   g_ref, be_ref, out_ref):
    agg = aggp_ref[0] + aggp_ref[1]
    cnt = cntp_ref[0] + cntp_ref[1]
    mean = agg / jnp.maximum(cnt, 1.0)
    dn = (((1,), (1,)), ((), ()))
    z = lax.dot_general(mean, wl_ref[...], dn,
                        preferred_element_type=jnp.float32)
    z = z + lax.dot_general(h_ref[...], wr_ref[...], dn,
                            preferred_element_type=jnp.float32)
    z = z + bl_ref[...]
    z = jnp.maximum(z, 0.0)
    mu = jnp.mean(z, axis=1, keepdims=True)
    d = z - mu
    var = jnp.mean(d * d, axis=1, keepdims=True)
    out_ref[...] = d * lax.rsqrt(var + 1e-5) * g_ref[...] + be_ref[...]


def _tc_dense(h, aggp, cntp, Wl, bl, Wr, g, be):
    grid = (N_NODES // ROWS_B,)
    return pl.pallas_call(
        _tc_dense_body,
        grid=grid,
        in_specs=[
            pl.BlockSpec((ROWS_B, D), lambda i: (i, 0)),
            pl.BlockSpec((NC, ROWS_B, D), lambda i: (0, i, 0)),
            pl.BlockSpec((NC, ROWS_B, 1), lambda i: (0, i, 0)),
            pl.BlockSpec((D, D), lambda i: (0, 0)),
            pl.BlockSpec((1, D), lambda i: (0, 0)),
            pl.BlockSpec((D, D), lambda i: (0, 0)),
            pl.BlockSpec((1, D), lambda i: (0, 0)),
            pl.BlockSpec((1, D), lambda i: (0, 0)),
        ],
        out_specs=pl.BlockSpec((ROWS_B, D), lambda i: (i, 0)),
        out_shape=jax.ShapeDtypeStruct((N_NODES, D), jnp.float32),
        name="sage_tc_dense",
    )(h, aggp, cntp, Wl, bl, Wr, g, be)


def kernel(x, edge_index, Wl0, bl0, Wr0, g0, be0, Wl1, bl1, Wr1, g1, be1):
    src = edge_index[0].astype(jnp.int32)
    dst = edge_index[1].astype(jnp.int32)
    packed = (src | (dst << 16)).reshape(NW, EPW)
    if EPW_PAD > EPW:
        pad_dst = (N_NODES + jnp.arange(EPW_PAD - EPW) % (N_ACC - N_NODES))
        pad_blk = jnp.broadcast_to(pad_dst << 16, (NW, EPW_PAD - EPW))
        packed = jnp.concatenate([packed, pad_blk], axis=1)
    packed = packed.reshape(NW, CH, K)

    aggp0, cntp = _sc_agg_with_cnt(x, packed)
    cnt3 = cntp.reshape(NC, N_PAD)[:, :N_NODES, None]
    agg3 = aggp0.reshape(NC, N_NODES, D)

    bl0r, g0r, be0r = bl0.reshape(1, D), g0.reshape(1, D), be0.reshape(1, D)
    bl1r, g1r, be1r = bl1.reshape(1, D), g1.reshape(1, D), be1.reshape(1, D)

    h1 = _tc_dense(x, agg3, cnt3, Wl0, bl0r, Wr0, g0r, be0r)
    aggp1 = _sc_agg_no_cnt(h1, packed)
    out = _tc_dense(h1, aggp1.reshape(NC, N_NODES, D), cnt3,
                    Wl1, bl1r, Wr1, g1r, be1r)
    return out


# speedup vs baseline: 1.0426x; 1.0426x over previous
import functools

import jax
import jax.numpy as jnp
from jax import lax
from jax.experimental import pallas as pl
from jax.experimental.pallas import tpu as pltpu
from jax.experimental.pallas import tpu_sc as plsc

N_NODES = 10000
D = 128
N_EDGES = 320000

NC = 2
NS = 16
NW = NC * NS
EPW = N_EDGES // NW
K = 64
CH = -(-EPW // K)
EPW_PAD = CH * K
N_ACC = 10240
PAD_DST = N_ACC - 1
NZT = 10
NROW_T = N_NODES // NZT
N_PAD = 10240
CPT = N_PAD // NS


def _sc_agg_body(with_cnt, *refs):
    if with_cnt:
        (h_hbm, pck3, outp, outc,
         acc_sh, cnt_sh, pck_all, src_a, dst_a, src_b, dst_b, src_c, dst_c,
         rows_a, rows_b, rows_c, ones_v, cbuf,
         sem_a, sem_b, sem_c) = refs
    else:
        (h_hbm, pck3, outp,
         acc_sh, pck_all, src_a, dst_a, src_b, dst_b, src_c, dst_c,
         rows_a, rows_b, rows_c,
         sem_a, sem_b, sem_c) = refs
    c = lax.axis_index("c")
    s = lax.axis_index("s")
    w = c * NS + s

    pltpu.sync_copy(pck3.at[w], pck_all)

    z16 = jnp.zeros((16,), jnp.float32)
    m16 = jnp.full((16,), 0xFFFF, jnp.int32)
    s16 = jnp.full((16,), 16, jnp.int32)

    def unpack(kk, src_v, dst_v):
        for j in range(K // 16):
            v = pck_all[kk, pl.ds(j * 16, 16)]
            src_v[pl.ds(j * 16, 16)] = v & m16
            dst_v[pl.ds(j * 16, 16)] = lax.shift_right_logical(v, s16)

    def _zrow(i, _):
        for j in range(8):
            rows_a[i, pl.ds(j * 16, 16)] = z16
        return 0
    lax.fori_loop(0, K, _zrow, 0)
    r0 = s * NROW_T

    @pl.when(s < NZT)
    def _zero_acc():
        for t in range(NROW_T // K):
            pltpu.sync_copy(rows_a, acc_sh.at[pl.ds(r0 + t * K, K)])
        pltpu.sync_copy(rows_a, acc_sh.at[pl.ds(r0 + NROW_T - K, K)])

    @pl.when(s == NZT)
    def _zero_garbage():
        for t in range((N_ACC - N_NODES) // K):
            pltpu.sync_copy(
                rows_a, acc_sh.at[pl.ds(N_NODES + t * K, K)])
        pltpu.sync_copy(rows_a, acc_sh.at[pl.ds(N_ACC - K, K)])

    if with_cnt:
        o16 = jnp.ones((16,), jnp.float32)
        for i in range(K // 16):
            ones_v[pl.ds(i * 16, 16)] = o16

        def _zc(i, _):
            cbuf[pl.ds(i * 16, 16)] = z16
            return 0
        lax.fori_loop(0, CPT // 16, _zc, 0)
        pltpu.sync_copy(cbuf, cnt_sh.at[pl.ds(s * CPT, CPT)])

    plsc.subcore_barrier()

    def g_start(src_v, buf, sem):
        pltpu.async_copy(h_hbm.at[src_v], buf, sem)

    def g_wait(src_v, buf, sem):
        pltpu.make_async_copy(h_hbm.at[src_v], buf, sem).wait()

    def s_add(dst_v, buf):
        pltpu.sync_copy(buf, acc_sh.at[dst_v], add=True)
        if with_cnt:
            pltpu.sync_copy(ones_v, cnt_sh.at[dst_v], add=True)

    B = ((src_a, dst_a, rows_a, sem_a),
         (src_b, dst_b, rows_b, sem_b),
         (src_c, dst_c, rows_c, sem_c))

    def step(m, start_next=True):
        sv, dv, buf, sem = B[m % 3]
        g_wait(sv, buf, sem)
        s_add(dv, buf)
        if start_next:
            nv, nd, nbuf, nsem = B[(m + 3) % 3]
            unpack(m + 3, nv, nd)
            g_start(nv, nbuf, nsem)

    unpack(0, src_a, dst_a)
    g_start(src_a, rows_a, sem_a)
    unpack(1, src_b, dst_b)
    g_start(src_b, rows_b, sem_b)
    unpack(2, src_c, dst_c)
    g_start(src_c, rows_c, sem_c)
    step(0)
    step(1)

    def _body(t, _):
        m0 = 2 + 3 * t
        for j in range(3):
            m = m0 + j
            sv, dv, buf, sem = B[(2 + j) % 3]
            g_wait(sv, buf, sem)
            s_add(dv, buf)
            nv, nd, nbuf, nsem = B[(2 + j) % 3]
            unpack(m + 3, nv, nd)
            g_start(nv, nbuf, nsem)
        return 0
    lax.fori_loop(0, (CH - 5) // 3, _body, 0)

    step(CH - 5)
    step(CH - 4)
    step(CH - 3, start_next=False)
    step(CH - 2, start_next=False)
    step(CH - 1, start_next=False)

    plsc.subcore_barrier()

    @pl.when(s < NZT)
    def _pub_acc():
        pltpu.sync_copy(acc_sh.at[pl.ds(r0, NROW_T)],
                        outp.at[pl.ds(c * N_NODES + r0, NROW_T)])
    if with_cnt:
        pltpu.sync_copy(cnt_sh.at[pl.ds(s * CPT, CPT)],
                        outc.at[pl.ds(c * N_PAD + s * CPT, CPT)])


def _make_sc_agg(with_cnt):
    out_type = [jax.ShapeDtypeStruct((NC * N_NODES, D), jnp.float32)]
    scratch = [
        pltpu.VMEM_SHARED((N_ACC, D), jnp.float32),
    ]
    if with_cnt:
        out_type.append(jax.ShapeDtypeStruct((NC * N_PAD,), jnp.float32))
        scratch.append(pltpu.VMEM_SHARED((N_PAD,), jnp.float32))
    scratch += [
        pltpu.VMEM((CH, K), jnp.int32),
        pltpu.VMEM((K,), jnp.int32),
        pltpu.VMEM((K,), jnp.int32),
        pltpu.VMEM((K,), jnp.int32),
        pltpu.VMEM((K,), jnp.int32),
        pltpu.VMEM((K,), jnp.int32),
        pltpu.VMEM((K,), jnp.int32),
        pltpu.VMEM((K, D), jnp.float32),
        pltpu.VMEM((K, D), jnp.float32),
        pltpu.VMEM((K, D), jnp.float32),
    ]
    if with_cnt:
        scratch.append(pltpu.VMEM((K,), jnp.float32))
        scratch.append(pltpu.VMEM((CPT,), jnp.float32))
    scratch += [pltpu.SemaphoreType.DMA] * 3
    mesh = plsc.VectorSubcoreMesh(core_axis_name="c", subcore_axis_name="s")
    return pl.kernel(
        functools.partial(_sc_agg_body, with_cnt),
        out_type=out_type if with_cnt else out_type[0],
        mesh=mesh,
        scratch_types=scratch,
        name="sage_sc_agg_cnt" if with_cnt else "sage_sc_agg",
    )


_sc_agg_with_cnt = _make_sc_agg(True)
_sc_agg_no_cnt = _make_sc_agg(False)

ROWS_B = 5000


def _tc_dense_body(h_ref, aggp_ref, cntp_ref, wl_ref, bl_ref, wr_ref,
                   g_ref, be_ref, out_ref):
    agg = aggp_ref[0] + aggp_ref[1]
    cnt = cntp_ref[0] + cntp_ref[1]
    mean = agg / jnp.maximum(cnt, 1.0)
    dn = (((1,), (1,)), ((), ()))
    z = lax.dot_general(mean, wl_ref[...], dn,
                        preferred_element_type=jnp.float32)
    z = z + lax.dot_general(h_ref[...], wr_ref[...], dn,
                            preferred_element_type=jnp.float32)
    z = z + bl_ref[...]
    z = jnp.maximum(z, 0.0)
    mu = jnp.mean(z, axis=1, keepdims=True)
    d = z - mu
    var = jnp.mean(d * d, axis=1, keepdims=True)
    out_ref[...] = d * lax.rsqrt(var + 1e-5) * g_ref[...] + be_ref[...]


def _tc_dense(h, aggp, cntp, Wl, bl, Wr, g, be):
    grid = (N_NODES // ROWS_B,)
    return pl.pallas_call(
        _tc_dense_body,
        grid=grid,
        in_specs=[
            pl.BlockSpec((ROWS_B, D), lambda i: (i, 0)),
            pl.BlockSpec((NC, ROWS_B, D), lambda i: (0, i, 0)),
            pl.BlockSpec((NC, ROWS_B, 1), lambda i: (0, i, 0)),
            pl.BlockSpec((D, D), lambda i: (0, 0)),
            pl.BlockSpec((1, D), lambda i: (0, 0)),
            pl.BlockSpec((D, D), lambda i: (0, 0)),
            pl.BlockSpec((1, D), lambda i: (0, 0)),
            pl.BlockSpec((1, D), lambda i: (0, 0)),
        ],
        out_specs=pl.BlockSpec((ROWS_B, D), lambda i: (i, 0)),
        out_shape=jax.ShapeDtypeStruct((N_NODES, D), jnp.float32),
        name="sage_tc_dense",
    )(h, aggp, cntp, Wl, bl, Wr, g, be)


def kernel(x, edge_index, Wl0, bl0, Wr0, g0, be0, Wl1, bl1, Wr1, g1, be1):
    src = edge_index[0].astype(jnp.int32)
    dst = edge_index[1].astype(jnp.int32)
    packed = (src | (dst << 16)).reshape(NW, EPW)
    if EPW_PAD > EPW:
        pad_dst = (N_NODES + jnp.arange(EPW_PAD - EPW) % (N_ACC - N_NODES))
        pad_blk = jnp.broadcast_to(pad_dst << 16, (NW, EPW_PAD - EPW))
        packed = jnp.concatenate([packed, pad_blk], axis=1)
    packed = packed.reshape(NW, CH, K)

    aggp0, cntp = _sc_agg_with_cnt(x, packed)
    cnt3 = cntp.reshape(NC, N_PAD)[:, :N_NODES, None]
    agg3 = aggp0.reshape(NC, N_NODES, D)

    bl0r, g0r, be0r = bl0.reshape(1, D), g0.reshape(1, D), be0.reshape(1, D)
    bl1r, g1r, be1r = bl1.reshape(1, D), g1.reshape(1, D), be1.reshape(1, D)

    h1 = _tc_dense(x, agg3, cnt3, Wl0, bl0r, Wr0, g0r, be0r)
    aggp1 = _sc_agg_no_cnt(h1, packed)
    out = _tc_dense(h1, aggp1.reshape(NC, N_NODES, D), cnt3,
                    Wl1, bl1r, Wr1, g1r, be1r)
    return out


# speedup vs baseline: 1.5489x; 1.4856x over previous
import functools

import jax
import jax.numpy as jnp
from jax import lax
from jax.experimental import pallas as pl
from jax.experimental.pallas import tpu as pltpu
from jax.experimental.pallas import tpu_sc as plsc

N_NODES = 10000
D = 128
N_EDGES = 320000

NC = 2
NS = 16
NW = NC * NS
EPW = N_EDGES // NW
K = 80
CH = -(-EPW // K)
EPW_PAD = CH * K
N_ACC = 10240
PAD_DST = N_ACC - 1
NZT = 10
NROW_T = N_NODES // NZT
N_PAD = 10240
CPT = N_PAD // NS


def _sc_agg_body(with_cnt, *refs):
    if with_cnt:
        (h_hbm, pckf, outp, outc,
         acc_sh, cnt_sh, src_a, dst_a, src_b, dst_b, src_c, dst_c,
         pck_a, pck_b, pck_c, rows_a, rows_b, rows_c, ones_v, cbuf,
         sem_a, sem_b, sem_c, sem_ia, sem_ib, sem_ic) = refs
    else:
        (h_hbm, pckf, outp,
         acc_sh, src_a, dst_a, src_b, dst_b, src_c, dst_c,
         pck_a, pck_b, pck_c, rows_a, rows_b, rows_c,
         sem_a, sem_b, sem_c, sem_ia, sem_ib, sem_ic) = refs
    c = lax.axis_index("c")
    s = lax.axis_index("s")
    w = c * NS + s
    e0 = w * EPW_PAD

    def i_start(kk, slot, sem):
        pltpu.async_copy(pckf.at[pl.ds(e0 + kk * K, K)], slot, sem)

    def i_wait(slot, sem):
        pltpu.make_async_copy(pckf.at[pl.ds(0, K)], slot, sem).wait()

    z16 = jnp.zeros((16,), jnp.float32)
    m16 = jnp.full((16,), 0xFFFF, jnp.int32)
    s16 = jnp.full((16,), 16, jnp.int32)

    def unpack(slot, src_v, dst_v):
        for j in range(K // 16):
            v = slot[pl.ds(j * 16, 16)]
            src_v[pl.ds(j * 16, 16)] = v & m16
            dst_v[pl.ds(j * 16, 16)] = lax.shift_right_logical(v, s16)

    def _zrow(i, _):
        for j in range(8):
            rows_a[i, pl.ds(j * 16, 16)] = z16
        return 0
    lax.fori_loop(0, K, _zrow, 0)
    r0 = s * NROW_T

    @pl.when(s < NZT)
    def _zero_acc():
        for t in range(NROW_T // K):
            pltpu.sync_copy(rows_a, acc_sh.at[pl.ds(r0 + t * K, K)])
        pltpu.sync_copy(rows_a, acc_sh.at[pl.ds(r0 + NROW_T - K, K)])

    @pl.when(s == NZT)
    def _zero_garbage():
        for t in range((N_ACC - N_NODES) // K):
            pltpu.sync_copy(
                rows_a, acc_sh.at[pl.ds(N_NODES + t * K, K)])
        pltpu.sync_copy(rows_a, acc_sh.at[pl.ds(N_ACC - K, K)])

    if with_cnt:
        o16 = jnp.ones((16,), jnp.float32)
        for i in range(K // 16):
            ones_v[pl.ds(i * 16, 16)] = o16

        def _zc(i, _):
            cbuf[pl.ds(i * 16, 16)] = z16
            return 0
        lax.fori_loop(0, CPT // 16, _zc, 0)
        pltpu.sync_copy(cbuf, cnt_sh.at[pl.ds(s * CPT, CPT)])

    plsc.subcore_barrier()

    def g_start(src_v, buf, sem):
        pltpu.async_copy(h_hbm.at[src_v], buf, sem)

    def g_wait(src_v, buf, sem):
        pltpu.make_async_copy(h_hbm.at[src_v], buf, sem).wait()

    def s_add(dst_v, buf):
        pltpu.sync_copy(buf, acc_sh.at[dst_v], add=True)
        if with_cnt:
            pltpu.sync_copy(ones_v, cnt_sh.at[dst_v], add=True)

    B = ((src_a, dst_a, pck_a, rows_a, sem_a, sem_ia),
         (src_b, dst_b, pck_b, rows_b, sem_b, sem_ib),
         (src_c, dst_c, pck_c, rows_c, sem_c, sem_ic))

    def step(m, kind, bidx=None):
        sv, dv, pv, buf, sg, si = B[bidx if bidx is not None else m % 3]
        g_wait(sv, buf, sg)
        s_add(dv, buf)
        if kind < 2:
            i_wait(pv, si)
            unpack(pv, sv, dv)
            g_start(sv, buf, sg)
            if kind == 0:
                i_start(m + 6, pv, si)

    for j in range(3):
        sv, dv, pv, buf, sg, si = B[j]
        i_start(j, pv, si)
    for j in range(3):
        sv, dv, pv, buf, sg, si = B[j]
        i_wait(pv, si)
        unpack(pv, sv, dv)
        g_start(sv, buf, sg)
        i_start(j + 3, pv, si)

    step(0, 0)
    step(1, 0)

    def _body(t, _):
        m0 = 2 + 3 * t
        for j in range(3):
            step(m0 + j, 0, bidx=(2 + j) % 3)
        return 0
    lax.fori_loop(0, (CH - 8) // 3, _body, 0)

    step(CH - 6, 1)
    step(CH - 5, 1)
    step(CH - 4, 1)
    step(CH - 3, 2)
    step(CH - 2, 2)
    step(CH - 1, 2)

    plsc.subcore_barrier()

    @pl.when(s < NZT)
    def _pub_acc():
        pltpu.sync_copy(acc_sh.at[pl.ds(r0, NROW_T)],
                        outp.at[pl.ds(c * N_NODES + r0, NROW_T)])
    if with_cnt:
        pltpu.sync_copy(cnt_sh.at[pl.ds(s * CPT, CPT)],
                        outc.at[pl.ds(c * N_PAD + s * CPT, CPT)])


def _make_sc_agg(with_cnt):
    out_type = [jax.ShapeDtypeStruct((NC * N_NODES, D), jnp.float32)]
    scratch = [
        pltpu.VMEM_SHARED((N_ACC, D), jnp.float32),
    ]
    if with_cnt:
        out_type.append(jax.ShapeDtypeStruct((NC * N_PAD,), jnp.float32))
        scratch.append(pltpu.VMEM_SHARED((N_PAD,), jnp.float32))
    scratch += [
        pltpu.VMEM((K,), jnp.int32),
        pltpu.VMEM((K,), jnp.int32),
        pltpu.VMEM((K,), jnp.int32),
        pltpu.VMEM((K,), jnp.int32),
        pltpu.VMEM((K,), jnp.int32),
        pltpu.VMEM((K,), jnp.int32),
        pltpu.VMEM((K,), jnp.int32),
        pltpu.VMEM((K,), jnp.int32),
        pltpu.VMEM((K,), jnp.int32),
        pltpu.VMEM((K, D), jnp.float32),
        pltpu.VMEM((K, D), jnp.float32),
        pltpu.VMEM((K, D), jnp.float32),
    ]
    if with_cnt:
        scratch.append(pltpu.VMEM((K,), jnp.float32))
        scratch.append(pltpu.VMEM((CPT,), jnp.float32))
    scratch += [pltpu.SemaphoreType.DMA] * 6
    mesh = plsc.VectorSubcoreMesh(core_axis_name="c", subcore_axis_name="s")
    return pl.kernel(
        functools.partial(_sc_agg_body, with_cnt),
        out_type=out_type if with_cnt else out_type[0],
        mesh=mesh,
        scratch_types=scratch,
        name="sage_sc_agg_cnt" if with_cnt else "sage_sc_agg",
    )


_sc_agg_with_cnt = _make_sc_agg(True)
_sc_agg_no_cnt = _make_sc_agg(False)

ROWS_B = 5000


def _tc_dense_body(h_ref, aggp_ref, cntp_ref, wl_ref, bl_ref, wr_ref,
                   g_ref, be_ref, out_ref):
    agg = aggp_ref[0] + aggp_ref[1]
    cnt = cntp_ref[0] + cntp_ref[1]
    mean = agg / jnp.maximum(cnt, 1.0)
    dn = (((1,), (1,)), ((), ()))
    z = lax.dot_general(mean, wl_ref[...], dn,
                        preferred_element_type=jnp.float32)
    z = z + lax.dot_general(h_ref[...], wr_ref[...], dn,
                            preferred_element_type=jnp.float32)
    z = z + bl_ref[...]
    z = jnp.maximum(z, 0.0)
    mu = jnp.mean(z, axis=1, keepdims=True)
    d = z - mu
    var = jnp.mean(d * d, axis=1, keepdims=True)
    out_ref[...] = d * lax.rsqrt(var + 1e-5) * g_ref[...] + be_ref[...]


def _tc_dense(h, aggp, cntp, Wl, bl, Wr, g, be):
    grid = (N_NODES // ROWS_B,)
    return pl.pallas_call(
        _tc_dense_body,
        grid=grid,
        in_specs=[
            pl.BlockSpec((ROWS_B, D), lambda i: (i, 0)),
            pl.BlockSpec((NC, ROWS_B, D), lambda i: (0, i, 0)),
            pl.BlockSpec((NC, ROWS_B, 1), lambda i: (0, i, 0)),
            pl.BlockSpec((D, D), lambda i: (0, 0)),
            pl.BlockSpec((1, D), lambda i: (0, 0)),
            pl.BlockSpec((D, D), lambda i: (0, 0)),
            pl.BlockSpec((1, D), lambda i: (0, 0)),
            pl.BlockSpec((1, D), lambda i: (0, 0)),
        ],
        out_specs=pl.BlockSpec((ROWS_B, D), lambda i: (i, 0)),
        out_shape=jax.ShapeDtypeStruct((N_NODES, D), jnp.float32),
        name="sage_tc_dense",
    )(h, aggp, cntp, Wl, bl, Wr, g, be)


def kernel(x, edge_index, Wl0, bl0, Wr0, g0, be0, Wl1, bl1, Wr1, g1, be1):
    src = edge_index[0].astype(jnp.int32)
    dst = edge_index[1].astype(jnp.int32)
    packed = (src | (dst << 16)).reshape(NW, EPW)
    if EPW_PAD > EPW:
        pad_dst = (N_NODES + jnp.arange(EPW_PAD - EPW) % (N_ACC - N_NODES))
        pad_blk = jnp.broadcast_to(pad_dst << 16, (NW, EPW_PAD - EPW))
        packed = jnp.concatenate([packed, pad_blk], axis=1)
    packed = packed.reshape(NW * CH * K)

    aggp0, cntp = _sc_agg_with_cnt(x, packed)
    cnt3 = cntp.reshape(NC, N_PAD)[:, :N_NODES, None]
    agg3 = aggp0.reshape(NC, N_NODES, D)

    bl0r, g0r, be0r = bl0.reshape(1, D), g0.reshape(1, D), be0.reshape(1, D)
    bl1r, g1r, be1r = bl1.reshape(1, D), g1.reshape(1, D), be1.reshape(1, D)

    h1 = _tc_dense(x, agg3, cnt3, Wl0, bl0r, Wr0, g0r, be0r)
    aggp1 = _sc_agg_no_cnt(h1, packed)
    out = _tc_dense(h1, aggp1.reshape(NC, N_NODES, D), cnt3,
                    Wl1, bl1r, Wr1, g1r, be1r)
    return out
